# baseline (speedup 1.0000x reference)
import functools

import jax
import jax.numpy as jnp
from jax import lax
from jax.experimental import pallas as pl
from jax.experimental.pallas import tpu as pltpu
from jax.experimental.pallas import tpu_sc as plsc

S, D, E, F, K = 2048, 1024, 8, 2048, 2
T = 192
NB = -(-(S * K) // T) + E
R = NB * T

NW = 32
PPW = (S * K) // NW
CH = 32
NCH = PPW // CH
TPW = S // NW
L = 16


_G = 128
_NG = S // _G


def _router_body(x_ref, wr_ref, gate_ref, pos_ref, bexp_ref, bval_ref):
    x = x_ref[...]
    logits = jnp.dot(x, wr_ref[...], preferred_element_type=jnp.float32)
    lane = lax.broadcasted_iota(jnp.int32, (S, E), 1)
    m1 = jnp.max(logits, axis=1, keepdims=True)
    i1 = jnp.argmax(logits, axis=1)[:, None]
    masked = jnp.where(lane == i1, -jnp.inf, logits)
    m2 = jnp.max(masked, axis=1, keepdims=True)
    i2 = jnp.argmax(masked, axis=1)[:, None]
    d = jnp.exp(m2 - m1)
    g1 = 1.0 / (1.0 + d)
    g2 = d / (1.0 + d)
    gate_ref[...] = jnp.concatenate([g1, g2], axis=1)

    oh1 = (lane == i1).astype(jnp.float32)
    oh2 = (lane == i2).astype(jnp.float32)
    gr = lax.broadcasted_iota(jnp.int32, (_G, _G), 0)
    gc = lax.broadcasted_iota(jnp.int32, (_G, _G), 1)
    l_incl = (gr >= gc).astype(jnp.float32)
    tots1 = []
    tots2 = []
    for g in range(_NG):
        sl = slice(g * _G, (g + 1) * _G)
        tots1.append(jnp.sum(oh1[sl, :], axis=0, keepdims=True))
        tots2.append(jnp.sum(oh2[sl, :], axis=0, keepdims=True))
    gt1 = jnp.concatenate(tots1, axis=0)
    gt2 = jnp.concatenate(tots2, axis=0)
    sr = lax.broadcasted_iota(jnp.int32, (_NG, _NG), 0)
    sc = lax.broadcasted_iota(jnp.int32, (_NG, _NG), 1)
    l_strict = (sr > sc).astype(jnp.float32)
    off1 = jnp.dot(l_strict, gt1, preferred_element_type=jnp.float32)
    off2 = jnp.dot(l_strict, gt2, preferred_element_type=jnp.float32)
    tot1 = jnp.sum(gt1, axis=0, keepdims=True)
    counts = tot1 + jnp.sum(gt2, axis=0, keepdims=True)

    cnt_i = counts.astype(jnp.int32)
    nblk = (cnt_i + (T - 1)) // T
    nblk_f = nblk.astype(jnp.float32)
    er = lax.broadcasted_iota(jnp.int32, (E, E), 0)
    ec = lax.broadcasted_iota(jnp.int32, (E, E), 1)
    bb = jnp.dot(nblk_f, (er < ec).astype(jnp.float32),
                 preferred_element_type=jnp.float32)

    p1 = []
    p2 = []
    for g in range(_NG):
        sl = slice(g * _G, (g + 1) * _G)
        o1g = oh1[sl, :]
        o2g = oh2[sl, :]
        c1g = off1[g:g + 1, :] + jnp.dot(
            l_incl, o1g, preferred_element_type=jnp.float32)
        c2g = tot1 + off2[g:g + 1, :] + jnp.dot(
            l_incl, o2g, preferred_element_type=jnp.float32)
        r1 = jnp.sum((bb * T + c1g - 1.0) * o1g, axis=1, keepdims=True)
        r2 = jnp.sum((bb * T + c2g - 1.0) * o2g, axis=1, keepdims=True)
        p1.append(r1)
        p2.append(r2)
    pos1 = jnp.concatenate(p1, axis=0)
    pos2 = jnp.concatenate(p2, axis=0)
    pos_ref[...] = jnp.concatenate([pos1, pos2], axis=1).astype(jnp.int32)

    bb_col = jnp.sum(jnp.where(ec < er, nblk_f, 0.0), axis=1,
                     keepdims=True)
    bid_en = lax.broadcasted_iota(jnp.int32, (E, NB), 1)
    bexp = jnp.sum((bb_col <= bid_en.astype(jnp.float32)).astype(jnp.int32),
                   axis=0, keepdims=True) - 1
    bexp_ref[...] = bexp
    used = jnp.sum(nblk, axis=1, keepdims=True)
    bid_1n = lax.broadcasted_iota(jnp.int32, (1, NB), 1)
    bval_ref[...] = (bid_1n < used).astype(jnp.int32)


def _router(x2d, wr):
    return pl.pallas_call(
        _router_body,
        out_shape=(
            jax.ShapeDtypeStruct((S, K), jnp.float32),
            jax.ShapeDtypeStruct((S, K), jnp.int32),
            jax.ShapeDtypeStruct((1, NB), jnp.int32),
            jax.ShapeDtypeStruct((1, NB), jnp.int32),
        ),
    )(x2d, wr)


def _dispatch_body(x_hbm, tok_hbm, pos_hbm, xs_hbm,
                   idx_t, idx_p, rows_a, rows_b, sg_a, sg_b, ss_a, ss_b):
    wid = lax.axis_index("s") * 2 + lax.axis_index("c")
    pltpu.sync_copy(tok_hbm.at[wid], idx_t)
    pltpu.sync_copy(pos_hbm.at[wid], idx_p)
    rows = (rows_a, rows_b)
    sg = (sg_a, sg_b)
    ss = (ss_a, ss_b)
    h_g = [None, None]
    h_s = [None, None]
    h_g[0] = pltpu.async_copy(x_hbm.at[idx_t.at[0]], rows[0], sg[0])
    for c in range(NCH):
        sl = c % 2
        if c + 1 < NCH:
            nsl = 1 - sl
            if h_s[nsl] is not None:
                h_s[nsl].wait()
            h_g[nsl] = pltpu.async_copy(
                x_hbm.at[idx_t.at[c + 1]], rows[nsl], sg[nsl])
        h_g[sl].wait()
        h_s[sl] = pltpu.async_copy(rows[sl], xs_hbm.at[idx_p.at[c]], ss[sl])
    h_s[0].wait()
    h_s[1].wait()


def _dispatch(x2d, tok3, pos3):
    mesh = plsc.VectorSubcoreMesh(core_axis_name="c", subcore_axis_name="s")
    return pl.kernel(
        _dispatch_body,
        out_type=jax.ShapeDtypeStruct((R, D), jnp.float32),
        mesh=mesh,
        scratch_types=[
            pltpu.VMEM((NCH, CH), jnp.int32),
            pltpu.VMEM((NCH, CH), jnp.int32),
            pltpu.VMEM((CH, D), jnp.float32),
            pltpu.VMEM((CH, D), jnp.float32),
            pltpu.SemaphoreType.DMA,
            pltpu.SemaphoreType.DMA,
            pltpu.SemaphoreType.DMA,
            pltpu.SemaphoreType.DMA,
        ],
    )(x2d, tok3, pos3)


def _ffn_body(be_ref, bv_ref, x_ref, w1_ref, b1_ref, w2_ref, b2_ref, o_ref):
    del be_ref
    b = pl.program_id(0)

    @pl.when(bv_ref[b] > 0)
    def _():
        h = jnp.dot(x_ref[...], w1_ref[0], preferred_element_type=jnp.float32)
        h = jax.nn.gelu(h + b1_ref[0])
        y = jnp.dot(h, w2_ref[0], preferred_element_type=jnp.float32)
        o_ref[...] = y + b2_ref[0]


def _ffn(x_sorted, W1, b1, W2, b2, block_expert, block_valid):
    grid_spec = pltpu.PrefetchScalarGridSpec(
        num_scalar_prefetch=2,
        grid=(NB,),
        in_specs=[
            pl.BlockSpec((T, D), lambda b, be, bv: (b, 0)),
            pl.BlockSpec((1, D, F), lambda b, be, bv: (be[b], 0, 0)),
            pl.BlockSpec((1, 1, F), lambda b, be, bv: (be[b], 0, 0)),
            pl.BlockSpec((1, F, D), lambda b, be, bv: (be[b], 0, 0)),
            pl.BlockSpec((1, 1, D), lambda b, be, bv: (be[b], 0, 0)),
        ],
        out_specs=pl.BlockSpec((T, D), lambda b, be, bv: (b, 0)),
    )
    return pl.pallas_call(
        _ffn_body,
        grid_spec=grid_spec,
        out_shape=jax.ShapeDtypeStruct((R, D), jnp.float32),
    )(block_expert, block_valid, x_sorted, W1,
      b1.reshape(E, 1, F), W2, b2.reshape(E, 1, D))


def _combine_body(y_hbm, pos_hbm, g_hbm, out_hbm,
                  idx_p, gall, rows_a, rows_b, obuf, sg_a, sg_b):
    wid = lax.axis_index("s") * 2 + lax.axis_index("c")
    pltpu.sync_copy(pos_hbm.at[wid], idx_p)
    pltpu.sync_copy(g_hbm.at[pl.ds(wid * PPW, PPW)], gall)
    rows = (rows_a, rows_b)
    sg = (sg_a, sg_b)
    h_g = [None, None]
    h_g[0] = pltpu.async_copy(y_hbm.at[idx_p.at[0]], rows[0], sg[0])
    for c in range(NCH):
        sl = c % 2
        if c + 1 < NCH:
            h_g[1 - sl] = pltpu.async_copy(
                y_hbm.at[idx_p.at[c + 1]], rows[1 - sl], sg[1 - sl])
        h_g[sl].wait()
        rbuf = rows[sl]
        gva = gall[pl.ds(c * CH, L)]
        gvb = gall[pl.ds(c * CH + L, L)]
        gs = ([gva[k] for k in range(L)] + [gvb[k] for k in range(L)])

        def body(j, _, gs=gs, rbuf=rbuf):
            sl_ = pl.ds(j * L, L)
            for i in range(CH // 2):
                a = rbuf[2 * i, sl_]
                b = rbuf[2 * i + 1, sl_]
                obuf[i, sl_] = gs[2 * i] * a + gs[2 * i + 1] * b
            return 0

        lax.fori_loop(0, D // L, body, 0)
        toff = wid * TPW + c * (CH // 2)
        pltpu.sync_copy(obuf, out_hbm.at[pl.ds(toff, CH // 2)])


def _combine(y, pos3, g_flat):
    mesh = plsc.VectorSubcoreMesh(core_axis_name="c", subcore_axis_name="s")
    return pl.kernel(
        _combine_body,
        out_type=jax.ShapeDtypeStruct((S, D), jnp.float32),
        mesh=mesh,
        scratch_types=[
            pltpu.VMEM((NCH, CH), jnp.int32),
            pltpu.VMEM((PPW,), jnp.float32),
            pltpu.VMEM((CH, D), jnp.float32),
            pltpu.VMEM((CH, D), jnp.float32),
            pltpu.VMEM((CH // 2, D), jnp.float32),
            pltpu.SemaphoreType.DMA,
            pltpu.SemaphoreType.DMA,
        ],
    )(y, pos3, g_flat)


def kernel(x, W_router, W1, b1, W2, b2):
    x2d = x.reshape(S, D)
    gates, pos, bexp, bval = _router(x2d, W_router)
    block_expert = bexp.reshape(NB)
    block_valid = bval.reshape(NB)

    tok = jnp.arange(S * K, dtype=jnp.int32) // K
    tok3 = tok.reshape(NW, NCH, CH)
    pos3 = pos.reshape(NW, NCH, CH)

    x_sorted = _dispatch(x2d, tok3, pos3)
    y = _ffn(x_sorted, W1, b1, W2, b2, block_expert, block_valid)
    out = _combine(y, pos3, gates.reshape(S * K))
    return out.reshape(1, S, D)


# speedup vs baseline: 1.0157x; 1.0157x over previous
import functools

import jax
import jax.numpy as jnp
from jax import lax
from jax.experimental import pallas as pl
from jax.experimental.pallas import tpu as pltpu
from jax.experimental.pallas import tpu_sc as plsc

S, D, E, F, K = 2048, 1024, 8, 2048, 2
T = 256
NB = -(-(S * K) // T) + E
R = NB * T

NW = 32
PPW = (S * K) // NW
CH = 32
NCH = PPW // CH
TPW = S // NW
L = 16


_G = 128
_NG = S // _G


def _router_body(x_ref, wr_ref, gate_ref, pos_ref, bexp_ref, bval_ref):
    x = x_ref[...]
    logits = jnp.dot(x, wr_ref[...], preferred_element_type=jnp.float32)
    lane = lax.broadcasted_iota(jnp.int32, (S, E), 1)
    m1 = jnp.max(logits, axis=1, keepdims=True)
    i1 = jnp.argmax(logits, axis=1)[:, None]
    masked = jnp.where(lane == i1, -jnp.inf, logits)
    m2 = jnp.max(masked, axis=1, keepdims=True)
    i2 = jnp.argmax(masked, axis=1)[:, None]
    d = jnp.exp(m2 - m1)
    g1 = 1.0 / (1.0 + d)
    g2 = d / (1.0 + d)
    gate_ref[...] = jnp.concatenate([g1, g2], axis=1)

    oh1 = (lane == i1).astype(jnp.float32)
    oh2 = (lane == i2).astype(jnp.float32)
    gr = lax.broadcasted_iota(jnp.int32, (_G, _G), 0)
    gc = lax.broadcasted_iota(jnp.int32, (_G, _G), 1)
    l_incl = (gr >= gc).astype(jnp.float32)
    tots1 = []
    tots2 = []
    for g in range(_NG):
        sl = slice(g * _G, (g + 1) * _G)
        tots1.append(jnp.sum(oh1[sl, :], axis=0, keepdims=True))
        tots2.append(jnp.sum(oh2[sl, :], axis=0, keepdims=True))
    gt1 = jnp.concatenate(tots1, axis=0)
    gt2 = jnp.concatenate(tots2, axis=0)
    sr = lax.broadcasted_iota(jnp.int32, (_NG, _NG), 0)
    sc = lax.broadcasted_iota(jnp.int32, (_NG, _NG), 1)
    l_strict = (sr > sc).astype(jnp.float32)
    off1 = jnp.dot(l_strict, gt1, preferred_element_type=jnp.float32)
    off2 = jnp.dot(l_strict, gt2, preferred_element_type=jnp.float32)
    tot1 = jnp.sum(gt1, axis=0, keepdims=True)
    counts = tot1 + jnp.sum(gt2, axis=0, keepdims=True)

    cnt_i = counts.astype(jnp.int32)
    nblk = (cnt_i + (T - 1)) // T
    nblk_f = nblk.astype(jnp.float32)
    er = lax.broadcasted_iota(jnp.int32, (E, E), 0)
    ec = lax.broadcasted_iota(jnp.int32, (E, E), 1)
    bb = jnp.dot(nblk_f, (er < ec).astype(jnp.float32),
                 preferred_element_type=jnp.float32)

    p1 = []
    p2 = []
    for g in range(_NG):
        sl = slice(g * _G, (g + 1) * _G)
        o1g = oh1[sl, :]
        o2g = oh2[sl, :]
        c1g = off1[g:g + 1, :] + jnp.dot(
            l_incl, o1g, preferred_element_type=jnp.float32)
        c2g = tot1 + off2[g:g + 1, :] + jnp.dot(
            l_incl, o2g, preferred_element_type=jnp.float32)
        r1 = jnp.sum((bb * T + c1g - 1.0) * o1g, axis=1, keepdims=True)
        r2 = jnp.sum((bb * T + c2g - 1.0) * o2g, axis=1, keepdims=True)
        p1.append(r1)
        p2.append(r2)
    pos1 = jnp.concatenate(p1, axis=0)
    pos2 = jnp.concatenate(p2, axis=0)
    pos_ref[...] = jnp.concatenate([pos1, pos2], axis=1).astype(jnp.int32)

    bb_col = jnp.sum(jnp.where(ec < er, nblk_f, 0.0), axis=1,
                     keepdims=True)
    bid_en = lax.broadcasted_iota(jnp.int32, (E, NB), 1)
    bexp = jnp.sum((bb_col <= bid_en.astype(jnp.float32)).astype(jnp.int32),
                   axis=0, keepdims=True) - 1
    bexp_ref[...] = bexp
    used = jnp.sum(nblk, axis=1, keepdims=True)
    bid_1n = lax.broadcasted_iota(jnp.int32, (1, NB), 1)
    bval_ref[...] = (bid_1n < used).astype(jnp.int32)


def _router(x2d, wr):
    return pl.pallas_call(
        _router_body,
        out_shape=(
            jax.ShapeDtypeStruct((S, K), jnp.float32),
            jax.ShapeDtypeStruct((S, K), jnp.int32),
            jax.ShapeDtypeStruct((1, NB), jnp.int32),
            jax.ShapeDtypeStruct((1, NB), jnp.int32),
        ),
    )(x2d, wr)


def _dispatch_body(x_hbm, tok_hbm, pos_hbm, xs_hbm,
                   idx_t, idx_p, rows_a, rows_b, sg_a, sg_b, ss_a, ss_b):
    wid = lax.axis_index("s") * 2 + lax.axis_index("c")
    pltpu.sync_copy(tok_hbm.at[wid], idx_t)
    pltpu.sync_copy(pos_hbm.at[wid], idx_p)
    rows = (rows_a, rows_b)
    sg = (sg_a, sg_b)
    ss = (ss_a, ss_b)
    h_g = [None, None]
    h_s = [None, None]
    h_g[0] = pltpu.async_copy(x_hbm.at[idx_t.at[0]], rows[0], sg[0])
    for c in range(NCH):
        sl = c % 2
        if c + 1 < NCH:
            nsl = 1 - sl
            if h_s[nsl] is not None:
                h_s[nsl].wait()
            h_g[nsl] = pltpu.async_copy(
                x_hbm.at[idx_t.at[c + 1]], rows[nsl], sg[nsl])
        h_g[sl].wait()
        h_s[sl] = pltpu.async_copy(rows[sl], xs_hbm.at[idx_p.at[c]], ss[sl])
    h_s[0].wait()
    h_s[1].wait()


def _dispatch(x2d, tok3, pos3):
    mesh = plsc.VectorSubcoreMesh(core_axis_name="c", subcore_axis_name="s")
    return pl.kernel(
        _dispatch_body,
        out_type=jax.ShapeDtypeStruct((R, D), jnp.float32),
        mesh=mesh,
        scratch_types=[
            pltpu.VMEM((NCH, CH), jnp.int32),
            pltpu.VMEM((NCH, CH), jnp.int32),
            pltpu.VMEM((CH, D), jnp.float32),
            pltpu.VMEM((CH, D), jnp.float32),
            pltpu.SemaphoreType.DMA,
            pltpu.SemaphoreType.DMA,
            pltpu.SemaphoreType.DMA,
            pltpu.SemaphoreType.DMA,
        ],
    )(x2d, tok3, pos3)


def _ffn_body(be_ref, bv_ref, x_ref, w1_ref, b1_ref, w2_ref, b2_ref, o_ref):
    del be_ref
    b = pl.program_id(0)

    @pl.when(bv_ref[b] > 0)
    def _():
        h = jnp.dot(x_ref[...], w1_ref[0], preferred_element_type=jnp.float32)
        h = jax.nn.gelu(h + b1_ref[0])
        y = jnp.dot(h, w2_ref[0], preferred_element_type=jnp.float32)
        o_ref[...] = y + b2_ref[0]


def _ffn(x_sorted, W1, b1, W2, b2, block_expert, block_valid):
    grid_spec = pltpu.PrefetchScalarGridSpec(
        num_scalar_prefetch=2,
        grid=(NB,),
        in_specs=[
            pl.BlockSpec((T, D), lambda b, be, bv: (b, 0)),
            pl.BlockSpec((1, D, F), lambda b, be, bv: (be[b], 0, 0)),
            pl.BlockSpec((1, 1, F), lambda b, be, bv: (be[b], 0, 0)),
            pl.BlockSpec((1, F, D), lambda b, be, bv: (be[b], 0, 0)),
            pl.BlockSpec((1, 1, D), lambda b, be, bv: (be[b], 0, 0)),
        ],
        out_specs=pl.BlockSpec((T, D), lambda b, be, bv: (b, 0)),
    )
    return pl.pallas_call(
        _ffn_body,
        grid_spec=grid_spec,
        out_shape=jax.ShapeDtypeStruct((R, D), jnp.float32),
    )(block_expert, block_valid, x_sorted, W1,
      b1.reshape(E, 1, F), W2, b2.reshape(E, 1, D))


def _combine_body(y_hbm, pos_hbm, g_hbm, out_hbm,
                  idx_p, gall, rows_a, rows_b, obuf, sg_a, sg_b):
    wid = lax.axis_index("s") * 2 + lax.axis_index("c")
    pltpu.sync_copy(pos_hbm.at[wid], idx_p)
    pltpu.sync_copy(g_hbm.at[pl.ds(wid * PPW, PPW)], gall)
    rows = (rows_a, rows_b)
    sg = (sg_a, sg_b)
    h_g = [None, None]
    h_g[0] = pltpu.async_copy(y_hbm.at[idx_p.at[0]], rows[0], sg[0])
    for c in range(NCH):
        sl = c % 2
        if c + 1 < NCH:
            h_g[1 - sl] = pltpu.async_copy(
                y_hbm.at[idx_p.at[c + 1]], rows[1 - sl], sg[1 - sl])
        h_g[sl].wait()
        rbuf = rows[sl]
        gva = gall[pl.ds(c * CH, L)]
        gvb = gall[pl.ds(c * CH + L, L)]
        gs = ([gva[k] for k in range(L)] + [gvb[k] for k in range(L)])

        def body(j, _, gs=gs, rbuf=rbuf):
            sl_ = pl.ds(j * L, L)
            for i in range(CH // 2):
                a = rbuf[2 * i, sl_]
                b = rbuf[2 * i + 1, sl_]
                obuf[i, sl_] = gs[2 * i] * a + gs[2 * i + 1] * b
            return 0

        lax.fori_loop(0, D // L, body, 0)
        toff = wid * TPW + c * (CH // 2)
        pltpu.sync_copy(obuf, out_hbm.at[pl.ds(toff, CH // 2)])


def _combine(y, pos3, g_flat):
    mesh = plsc.VectorSubcoreMesh(core_axis_name="c", subcore_axis_name="s")
    return pl.kernel(
        _combine_body,
        out_type=jax.ShapeDtypeStruct((S, D), jnp.float32),
        mesh=mesh,
        scratch_types=[
            pltpu.VMEM((NCH, CH), jnp.int32),
            pltpu.VMEM((PPW,), jnp.float32),
            pltpu.VMEM((CH, D), jnp.float32),
            pltpu.VMEM((CH, D), jnp.float32),
            pltpu.VMEM((CH // 2, D), jnp.float32),
            pltpu.SemaphoreType.DMA,
            pltpu.SemaphoreType.DMA,
        ],
    )(y, pos3, g_flat)


def kernel(x, W_router, W1, b1, W2, b2):
    x2d = x.reshape(S, D)
    gates, pos, bexp, bval = _router(x2d, W_router)
    block_expert = bexp.reshape(NB)
    block_valid = bval.reshape(NB)

    tok = jnp.arange(S * K, dtype=jnp.int32) // K
    tok3 = tok.reshape(NW, NCH, CH)
    pos3 = pos.reshape(NW, NCH, CH)

    x_sorted = _dispatch(x2d, tok3, pos3)
    y = _ffn(x_sorted, W1, b1, W2, b2, block_expert, block_valid)
    out = _combine(y, pos3, gates.reshape(S * K))
    return out.reshape(1, S, D)
